# Initial kernel scaffold; baseline (speedup 1.0000x reference)
import jax
import jax.numpy as jnp
from jax.experimental import pallas as pl


def kernel(x, Wq, bq, Wk, bk, Wv, bv, part_emb, Wo, bo):
    raise NotImplementedError("write your pallas kernel here")


# speedup vs baseline: 13.8209x; 13.8209x over previous
import numpy as np
import jax
import jax.numpy as jnp
from jax import lax
from jax.experimental import pallas as pl
from jax.experimental.pallas import tpu as pltpu

_R = 16


def _attn_kernel(xh_ref, wq_ref, bq_ref, wk_ref, bk_ref, wv_ref, bv_ref,
                 pe_ref, wot_ref, bo_ref, out_ref):
    h = pl.program_id(0)
    S, HD = xh_ref.shape[1], xh_ref.shape[2]
    P = pe_ref.shape[1]
    PR = P * _R

    x = xh_ref[0]
    q = jnp.dot(x, wq_ref[0], preferred_element_type=jnp.float32) + bq_ref[0]
    kk = jnp.dot(x, wk_ref[0], preferred_element_type=jnp.float32) + bk_ref[0]
    vv = jnp.dot(x, wv_ref[0], preferred_element_type=jnp.float32) + bv_ref[0]

    logits = lax.dot_general(q, pe_ref[0], (((1,), (1,)), ((), ())),
                             preferred_element_type=jnp.float32)
    pid = lax.broadcasted_iota(jnp.int32, (S, P), 1)
    m1 = jnp.max(logits, axis=-1, keepdims=True)
    am1 = jnp.min(jnp.where(logits == m1, pid, P), axis=-1, keepdims=True)
    l2 = jnp.where(pid == am1, -jnp.inf, logits)
    m2 = jnp.max(l2, axis=-1, keepdims=True)
    am2 = jnp.min(jnp.where(l2 == m2, pid, P), axis=-1, keepdims=True)
    e2 = jnp.exp(m2 - m1)
    g1 = 1.0 / (1.0 + e2)
    g2 = e2 / (1.0 + e2)

    row = lax.broadcasted_iota(jnp.int32, (S, 1), 0) % _R
    col = lax.broadcasted_iota(jnp.int32, (S, PR), 1)
    cp = col // _R
    hit_r = (col % _R) == row
    A = (jnp.where((cp == am1) & hit_r, g1, 0.0)
         + jnp.where((cp == am2) & hit_r, g2, 0.0))

    st_k = lax.dot_general(A, kk, (((0,), (0,)), ((), ())),
                           preferred_element_type=jnp.float32)
    st_v = lax.dot_general(A, vv, (((0,), (0,)), ((), ())),
                           preferred_element_type=jnp.float32)

    scores = lax.dot_general(q, st_k, (((1,), (1,)), ((), ())),
                             preferred_element_type=jnp.float32)
    scores = scores * (1.0 / np.sqrt(HD))
    sel = (cp == am1) | (cp == am2)
    sm = jnp.where(sel, scores, -jnp.inf)
    mx = jnp.max(sm, axis=-1, keepdims=True)
    prob = jnp.where(sel, jnp.exp(sm - mx), 0.0)
    aw = prob / jnp.sum(prob, axis=-1, keepdims=True)
    hv = jnp.dot(aw, st_v, preferred_element_type=jnp.float32)

    contrib = jnp.dot(hv, wot_ref[0], preferred_element_type=jnp.float32)

    @pl.when(h == 0)
    def _():
        out_ref[...] = jnp.broadcast_to(bo_ref[...], out_ref.shape)

    out_ref[...] += contrib


def kernel(x, Wq, bq, Wk, bk, Wv, bv, part_emb, Wo, bo):
    B, S, D = x.shape
    H, HD, _ = Wq.shape
    P = part_emb.shape[1]

    xh = x.reshape(S, H, HD).transpose(1, 0, 2)
    wot = Wo.T.reshape(H, HD, D)
    bq3 = bq.reshape(H, 1, HD)
    bk3 = bk.reshape(H, 1, HD)
    bv3 = bv.reshape(H, 1, HD)
    bo2 = bo.reshape(1, D)

    head_spec = lambda shape: pl.BlockSpec(shape, lambda h: (h,) + (0,) * (len(shape) - 1))
    out = pl.pallas_call(
        _attn_kernel,
        grid=(H,),
        in_specs=[
            head_spec((1, S, HD)),
            head_spec((1, HD, HD)), head_spec((1, 1, HD)),
            head_spec((1, HD, HD)), head_spec((1, 1, HD)),
            head_spec((1, HD, HD)), head_spec((1, 1, HD)),
            head_spec((1, P, HD)),
            head_spec((1, HD, D)),
            pl.BlockSpec((1, D), lambda h: (0, 0)),
        ],
        out_specs=pl.BlockSpec((S, D), lambda h: (0, 0)),
        out_shape=jax.ShapeDtypeStruct((S, D), jnp.float32),
        compiler_params=pltpu.CompilerParams(
            dimension_semantics=("arbitrary",)),
    )(xh, Wq, bq3, Wk, bk3, Wv, bv3, part_emb, wot, bo2)
    return out.reshape(B, S, D)
